# baseline (speedup 1.0000x reference)
import functools
import jax
import jax.numpy as jnp
from jax import lax
from jax.experimental import pallas as pl
from jax.experimental.pallas import tpu as pltpu
from jax.experimental.pallas import tpu_sc as plsc

_NC = 2
_NS = 16
_NW = _NC * _NS
_V = 8192
_D = 8192
_BTOK = 8192
_BPW = _BTOK // _NW
_L = 16
_NCHUNK = _BPW // _L
_W = 16


def _make_gather():
  mesh = plsc.VectorSubcoreMesh(
      core_axis_name="c", subcore_axis_name="s",
      num_cores=_NC, num_subcores=_NS)

  @functools.partial(
      pl.kernel,
      out_type=jax.ShapeDtypeStruct((_BTOK, _D), jnp.float32),
      mesh=mesh,
      scratch_types=[
          pltpu.VMEM((_BPW,), jnp.int32),
          pltpu.SemaphoreType.DMA,
      ],
  )
  def gather(table_hbm, idx_hbm, out_hbm, idx_v, sem):
    wid = lax.axis_index("s") * _NC + lax.axis_index("c")
    row0 = wid * _BPW
    pltpu.sync_copy(idx_hbm.at[wid], idx_v)
    lane = lax.iota(jnp.int32, _L)

    @pl.loop(0, _NCHUNK)
    def _body(c):
      v = idx_v[pl.ds(c * _L, _L)]
      for j in range(_L):
        r = v[j]
        i = c * _L + j
        pltpu.async_copy(
            table_hbm.at[pl.ds(r, 1)],
            out_hbm.at[pl.ds(row0 + i, 1)], sem)

      @pl.when(c >= 1)
      def _slide():
        for _ in range(_L):
          pltpu.make_async_copy(
              table_hbm.at[pl.ds(0, 1)],
              out_hbm.at[pl.ds(row0, 1)], sem).wait()

    for _ in range(_W):
      pltpu.make_async_copy(
          table_hbm.at[pl.ds(0, 1)],
          out_hbm.at[pl.ds(row0, 1)], sem).wait()

  return gather


_gather = _make_gather()


def kernel(idx, table):
  b, t = idx.shape
  idx2 = idx.astype(jnp.int32).reshape(_NW, _BPW)
  out = _gather(table, idx2)
  return out.reshape(b, t, _V)


# speedup vs baseline: 39.2953x; 39.2953x over previous
import functools

import jax
import jax.numpy as jnp
from jax import lax
from jax.experimental import pallas as pl
from jax.experimental.pallas import tpu as pltpu
from jax.experimental.pallas import tpu_sc as plsc

_NC = 2
_NS = 16
_NW = _NC * _NS

_V = 8192
_D = 8192
_BTOK = 8192
_BPW = _BTOK // _NW
_K = 2
_NBUF = 4
_LA = 2
_NCHUNK = _BPW // _K


def _make_gather():
  mesh = plsc.VectorSubcoreMesh(
      core_axis_name="c", subcore_axis_name="s",
      num_cores=_NC, num_subcores=_NS)

  @functools.partial(
      pl.kernel,
      out_type=jax.ShapeDtypeStruct((_BTOK, _D), jnp.float32),
      mesh=mesh,
      scratch_types=[
          pltpu.VMEM((_NCHUNK, _K), jnp.int32),
      ] + [pltpu.VMEM((_K, _D), jnp.float32)] * _NBUF
        + [pltpu.SemaphoreType.DMA] * (2 * _NBUF),
  )
  def gather(table_hbm, idx_hbm, out_hbm, idx_v, *rest):
    bufs = rest[:_NBUF]
    sem_in = rest[_NBUF:2 * _NBUF]
    sem_out = rest[2 * _NBUF:]
    wid = lax.axis_index("s") * _NC + lax.axis_index("c")
    row0 = wid * _BPW

    pltpu.sync_copy(idx_hbm.at[wid], idx_v)

    for j in range(_LA):
      pltpu.async_copy(table_hbm.at[idx_v.at[j]], bufs[j % _NBUF],
                       sem_in[j % _NBUF])

    @pl.loop(0, _NCHUNK, step=_NBUF)
    def _body(g):
      for b in range(_NBUF):
        c = g + b
        b2 = (b + _LA) % _NBUF

        @pl.when(c + _LA < _NCHUNK)
        def _refill():
          @pl.when(c + _LA - _NBUF >= 0)
          def _free():
            pltpu.make_async_copy(
                bufs[b2],
                out_hbm.at[pl.ds(row0 + (c + _LA - _NBUF) * _K, _K)],
                sem_out[b2]).wait()
          pltpu.async_copy(table_hbm.at[idx_v.at[c + _LA]], bufs[b2],
                           sem_in[b2])

        pltpu.make_async_copy(
            table_hbm.at[idx_v.at[c]], bufs[b], sem_in[b]).wait()
        pltpu.async_copy(bufs[b], out_hbm.at[pl.ds(row0 + c * _K, _K)],
                         sem_out[b])

    for b in range(_NBUF):
      c = _NCHUNK - _NBUF + b
      pltpu.make_async_copy(
          bufs[b], out_hbm.at[pl.ds(row0 + c * _K, _K)], sem_out[b]).wait()

  return gather


_gather = _make_gather()


def kernel(idx, table):
  b, t = idx.shape
  idx3 = idx.astype(jnp.int32).reshape(_NW, _NCHUNK, _K)
  out = _gather(table, idx3)
  return out.reshape(b, t, _V)


# speedup vs baseline: 39.5933x; 1.0076x over previous
import functools

import jax
import jax.numpy as jnp
from jax import lax
from jax.experimental import pallas as pl
from jax.experimental.pallas import tpu as pltpu
from jax.experimental.pallas import tpu_sc as plsc

_NC = 2
_NS = 16
_NW = _NC * _NS

_V = 8192
_D = 8192
_BTOK = 8192
_BPW = _BTOK // _NW
_K = 4
_NBUF = 2
_LA = 1
_NCHUNK = _BPW // _K


def _make_gather():
  mesh = plsc.VectorSubcoreMesh(
      core_axis_name="c", subcore_axis_name="s",
      num_cores=_NC, num_subcores=_NS)

  @functools.partial(
      pl.kernel,
      out_type=jax.ShapeDtypeStruct((_BTOK, _D), jnp.float32),
      mesh=mesh,
      scratch_types=[
          pltpu.VMEM((_NCHUNK, _K), jnp.int32),
      ] + [pltpu.VMEM((_K, _D), jnp.float32)] * _NBUF
        + [pltpu.SemaphoreType.DMA] * (2 * _NBUF),
  )
  def gather(table_hbm, idx_hbm, out_hbm, idx_v, *rest):
    bufs = rest[:_NBUF]
    sem_in = rest[_NBUF:2 * _NBUF]
    sem_out = rest[2 * _NBUF:]
    wid = lax.axis_index("s") * _NC + lax.axis_index("c")
    row0 = wid * _BPW

    pltpu.sync_copy(idx_hbm.at[wid], idx_v)

    for j in range(_LA):
      pltpu.async_copy(table_hbm.at[idx_v.at[j]], bufs[j % _NBUF],
                       sem_in[j % _NBUF])

    @pl.loop(0, _NCHUNK, step=_NBUF)
    def _body(g):
      for b in range(_NBUF):
        c = g + b
        b2 = (b + _LA) % _NBUF

        @pl.when(c + _LA < _NCHUNK)
        def _refill():
          @pl.when(c + _LA - _NBUF >= 0)
          def _free():
            pltpu.make_async_copy(
                bufs[b2],
                out_hbm.at[pl.ds(row0 + (c + _LA - _NBUF) * _K, _K)],
                sem_out[b2]).wait()
          pltpu.async_copy(table_hbm.at[idx_v.at[c + _LA]], bufs[b2],
                           sem_in[b2])

        pltpu.make_async_copy(
            table_hbm.at[idx_v.at[c]], bufs[b], sem_in[b]).wait()
        pltpu.async_copy(bufs[b], out_hbm.at[pl.ds(row0 + c * _K, _K)],
                         sem_out[b])

    for b in range(_NBUF):
      c = _NCHUNK - _NBUF + b
      pltpu.make_async_copy(
          bufs[b], out_hbm.at[pl.ds(row0 + c * _K, _K)], sem_out[b]).wait()

  return gather


_gather = _make_gather()


def kernel(idx, table):
  b, t = idx.shape
  idx3 = idx.astype(jnp.int32).reshape(_NW, _NCHUNK, _K)
  out = _gather(table, idx3)
  return out.reshape(b, t, _V)
